# baseline (speedup 1.0000x reference)
import functools

import jax
import jax.numpy as jnp
from jax import lax
from jax.experimental import pallas as pl
from jax.experimental.pallas import tpu as pltpu
from jax.experimental.pallas import tpu_sc as plsc

B, S = 4, 2048
D = 128
EPS = 1e-07

NC, NS = 2, 16
NW = NC * NS
NT = B * S
TPW = NT // NW
CHUNK = 64
NCH = TPW // CHUNK
NG = D // 16
UNROLL = 2

_GDNUMS = lax.GatherDimensionNumbers(
    offset_dims=(), collapsed_slice_dims=(0,), start_index_map=(0,))


def _allsum(v):
    for k in (8, 4, 2, 1):
        idx = lax.iota(jnp.int32, 16) ^ k
        p = lax.gather(v, idx[:, None], _GDNUMS, (1,),
                       mode=lax.GatherScatterMode.PROMISE_IN_BOUNDS)
        v = v + p
    return v


def _rsqrt(v):
    i = lax.bitcast_convert_type(v, jnp.int32)
    i = 0x5F3759DF - lax.shift_right_logical(i, 1)
    y = lax.bitcast_convert_type(i, jnp.float32)
    for _ in range(3):
        y = y * (1.5 - 0.5 * v * y * y)
    return y


def _body(ids_hbm, wt_hbm, pos_hbm, tt_hbm, g_hbm, b_hbm, out_hbm,
          idx_v, rows_v, pos_v, tt_v, g_v, b_v, sem_g, sem_o):
    c = lax.axis_index("c")
    s = lax.axis_index("s")
    wid = s * NC + c
    b = wid // (S // TPW)
    pbase = lax.rem(wid, S // TPW) * TPW

    pltpu.sync_copy(ids_hbm.at[b, pl.ds(pbase, TPW)], idx_v)
    gathers = [
        pltpu.async_copy(wt_hbm.at[idx_v.at[pl.ds(j * CHUNK, CHUNK)]],
                         rows_v.at[pl.ds(j * CHUNK, CHUNK)], sem_g.at[j])
        for j in range(NCH)
    ]
    pltpu.sync_copy(pos_hbm.at[pl.ds(pbase, TPW)], pos_v)
    pltpu.sync_copy(tt_hbm.at[pl.ds(0, 1)], tt_v)
    pltpu.sync_copy(g_hbm, g_v)
    pltpu.sync_copy(b_hbm, b_v)

    tt_g = [tt_v[0, pl.ds(16 * g, 16)] for g in range(NG)]
    ga_g = [g_v[pl.ds(16 * g, 16)] for g in range(NG)]
    be_g = [b_v[pl.ds(16 * g, 16)] for g in range(NG)]
    inv_d = 1.0 / D

    def ln_row(r):
        xs = []
        for g in range(NG):
            x = rows_v[r, pl.ds(16 * g, 16)] + pos_v[r, pl.ds(16 * g, 16)]
            xs.append(x + tt_g[g])
        s1 = ((xs[0] + xs[1]) + (xs[2] + xs[3])) + \
             ((xs[4] + xs[5]) + (xs[6] + xs[7]))
        qs = [x * x for x in xs]
        s2 = ((qs[0] + qs[1]) + (qs[2] + qs[3])) + \
             ((qs[4] + qs[5]) + (qs[6] + qs[7]))
        mean = _allsum(s1) * inv_d
        ex2 = _allsum(s2) * inv_d
        y = _rsqrt(ex2 - mean * mean + EPS)
        for g in range(NG):
            rows_v[r, pl.ds(16 * g, 16)] = \
                (xs[g] - mean) * y * ga_g[g] + be_g[g]

    outs = []
    for j in range(NCH):
        gathers[j].wait()

        def chunk_fn(it, carry, j=j):
            r0 = j * CHUNK + it * UNROLL
            for u in range(UNROLL):
                ln_row(r0 + u)
            return carry

        lax.fori_loop(0, CHUNK // UNROLL, chunk_fn, 0)
        outs.append(
            pltpu.async_copy(rows_v.at[pl.ds(j * CHUNK, CHUNK)],
                             out_hbm.at[b, pl.ds(pbase + j * CHUNK, CHUNK)],
                             sem_o))
    for cp in outs:
        cp.wait()


@jax.jit
def kernel(input_ids, word_table, pos_table, tt_table, gamma, beta):
    ids = input_ids.astype(jnp.int32)
    run = functools.partial(
        pl.kernel,
        out_type=jax.ShapeDtypeStruct((B, S, D), jnp.float32),
        mesh=plsc.VectorSubcoreMesh(core_axis_name="c", subcore_axis_name="s"),
        scratch_types=[
            pltpu.VMEM((TPW,), jnp.int32),
            pltpu.VMEM((TPW, D), jnp.float32),
            pltpu.VMEM((TPW, D), jnp.float32),
            pltpu.VMEM((1, D), jnp.float32),
            pltpu.VMEM((D,), jnp.float32),
            pltpu.VMEM((D,), jnp.float32),
            pltpu.SemaphoreType.DMA((NCH,)),
            pltpu.SemaphoreType.DMA,
        ],
    )(_body)
    return run(ids, word_table, pos_table, tt_table, gamma, beta)


# speedup vs baseline: 1.2284x; 1.2284x over previous
import functools

import jax
import jax.numpy as jnp
from jax import lax
from jax.experimental import pallas as pl
from jax.experimental.pallas import tpu as pltpu
from jax.experimental.pallas import tpu_sc as plsc

B, S = 4, 2048
D = 128
EPS = 1e-07

NC, NS = 2, 16
NW = NC * NS
NT = B * S
TPW = NT // NW
CHUNK = 128
NCH = TPW // CHUNK


def _gather_body(ids_hbm, wt_hbm, out_hbm, idx_v, rows_v, sem_g, sem_o):
    c = lax.axis_index("c")
    s = lax.axis_index("s")
    wid = s * NC + c
    base = wid * TPW
    b = wid // (S // TPW)
    pbase = lax.rem(wid, S // TPW) * TPW

    pltpu.sync_copy(ids_hbm.at[b, pl.ds(pbase, TPW)], idx_v)
    gathers = [
        pltpu.async_copy(wt_hbm.at[idx_v.at[pl.ds(j * CHUNK, CHUNK)]],
                         rows_v.at[pl.ds(j * CHUNK, CHUNK)], sem_g.at[j])
        for j in range(NCH)
    ]
    outs = []
    for j in range(NCH):
        gathers[j].wait()
        outs.append(
            pltpu.async_copy(rows_v.at[pl.ds(j * CHUNK, CHUNK)],
                             out_hbm.at[b, pl.ds(pbase + j * CHUNK, CHUNK)],
                             sem_o))
    for cp in outs:
        cp.wait()


def _sc_gather(ids, word_table):
    run = functools.partial(
        pl.kernel,
        out_type=jax.ShapeDtypeStruct((B, S, D), jnp.float32),
        mesh=plsc.VectorSubcoreMesh(core_axis_name="c", subcore_axis_name="s"),
        scratch_types=[
            pltpu.VMEM((TPW,), jnp.int32),
            pltpu.VMEM((TPW, D), jnp.float32),
            pltpu.SemaphoreType.DMA((NCH,)),
            pltpu.SemaphoreType.DMA,
        ],
    )(_gather_body)
    return run(ids, word_table)


BPB = 2


def _ln_body(rows_ref, pos_ref, tt_ref, g_ref, b_ref, o_ref):
    x = rows_ref[...] + pos_ref[...] + tt_ref[0:1, 0:1, :]
    mean = jnp.mean(x, axis=-1, keepdims=True)
    xc = x - mean
    var = jnp.mean(xc * xc, axis=-1, keepdims=True)
    o_ref[...] = xc * lax.rsqrt(var + EPS) * g_ref[0:1, 0:1, :] + b_ref[0:1, 0:1, :]


def _tc_layernorm(rows, pos_table, tt_table, gamma, beta):
    return pl.pallas_call(
        _ln_body,
        grid=(B // BPB,),
        in_specs=[
            pl.BlockSpec((BPB, S, D), lambda i: (i, 0, 0)),
            pl.BlockSpec((1, S, D), lambda i: (0, 0, 0)),
            pl.BlockSpec((1, 2, D), lambda i: (0, 0, 0)),
            pl.BlockSpec((1, 1, D), lambda i: (0, 0, 0)),
            pl.BlockSpec((1, 1, D), lambda i: (0, 0, 0)),
        ],
        out_specs=pl.BlockSpec((BPB, S, D), lambda i: (i, 0, 0)),
        out_shape=jax.ShapeDtypeStruct((B, S, D), jnp.float32),
    )(rows, pos_table.reshape(1, S, D),
      tt_table.reshape(1, 2, D), gamma, beta)


@jax.jit
def kernel(input_ids, word_table, pos_table, tt_table, gamma, beta):
    ids = input_ids.astype(jnp.int32)
    rows = _sc_gather(ids, word_table)
    return _tc_layernorm(rows, pos_table, tt_table,
                         gamma.reshape(1, 1, D), beta.reshape(1, 1, D))


# speedup vs baseline: 1.2331x; 1.0038x over previous
import functools

import jax
import jax.numpy as jnp
from jax import lax
from jax.experimental import pallas as pl
from jax.experimental.pallas import tpu as pltpu
from jax.experimental.pallas import tpu_sc as plsc

B, S = 4, 2048
D = 128
EPS = 1e-07

NC, NS = 2, 16
NW = NC * NS
NT = B * S
TPW = NT // NW
CHUNK = 128
NCH = TPW // CHUNK


def _gather_body(ids_hbm, wt_hbm, out_hbm, idx_v, rows_v, sem_g, sem_o):
    c = lax.axis_index("c")
    s = lax.axis_index("s")
    wid = s * NC + c
    base = wid * TPW
    b = wid // (S // TPW)
    pbase = lax.rem(wid, S // TPW) * TPW

    gathers = []
    for j in range(NCH):
        pltpu.sync_copy(ids_hbm.at[b, pl.ds(pbase + j * CHUNK, CHUNK)],
                        idx_v.at[pl.ds(j * CHUNK, CHUNK)])
        gathers.append(
            pltpu.async_copy(wt_hbm.at[idx_v.at[pl.ds(j * CHUNK, CHUNK)]],
                             rows_v.at[pl.ds(j * CHUNK, CHUNK)], sem_g.at[j]))
    outs = []
    for j in range(NCH):
        gathers[j].wait()
        outs.append(
            pltpu.async_copy(rows_v.at[pl.ds(j * CHUNK, CHUNK)],
                             out_hbm.at[b, pl.ds(pbase + j * CHUNK, CHUNK)],
                             sem_o))
    for cp in outs:
        cp.wait()


def _sc_gather(ids, word_table):
    run = functools.partial(
        pl.kernel,
        out_type=jax.ShapeDtypeStruct((B, S, D), jnp.float32),
        mesh=plsc.VectorSubcoreMesh(core_axis_name="c", subcore_axis_name="s"),
        scratch_types=[
            pltpu.VMEM((TPW,), jnp.int32),
            pltpu.VMEM((TPW, D), jnp.float32),
            pltpu.SemaphoreType.DMA((NCH,)),
            pltpu.SemaphoreType.DMA,
        ],
    )(_gather_body)
    return run(ids, word_table)


BPB = 2


def _ln_body(rows_ref, pos_ref, tt_ref, g_ref, b_ref, o_ref):
    x = rows_ref[...] + pos_ref[...] + tt_ref[0:1, 0:1, :]
    mean = jnp.mean(x, axis=-1, keepdims=True)
    xc = x - mean
    var = jnp.mean(xc * xc, axis=-1, keepdims=True)
    o_ref[...] = xc * lax.rsqrt(var + EPS) * g_ref[0:1, 0:1, :] + b_ref[0:1, 0:1, :]


def _tc_layernorm(rows, pos_table, tt_table, gamma, beta):
    return pl.pallas_call(
        _ln_body,
        grid=(B // BPB,),
        in_specs=[
            pl.BlockSpec((BPB, S, D), lambda i: (i, 0, 0)),
            pl.BlockSpec((1, S, D), lambda i: (0, 0, 0)),
            pl.BlockSpec((1, 2, D), lambda i: (0, 0, 0)),
            pl.BlockSpec((1, 1, D), lambda i: (0, 0, 0)),
            pl.BlockSpec((1, 1, D), lambda i: (0, 0, 0)),
        ],
        out_specs=pl.BlockSpec((BPB, S, D), lambda i: (i, 0, 0)),
        out_shape=jax.ShapeDtypeStruct((B, S, D), jnp.float32),
    )(rows, pos_table.reshape(1, S, D),
      tt_table.reshape(1, 2, D), gamma, beta)


@jax.jit
def kernel(input_ids, word_table, pos_table, tt_table, gamma, beta):
    ids = input_ids.astype(jnp.int32)
    rows = _sc_gather(ids, word_table)
    return _tc_layernorm(rows, pos_table, tt_table,
                         gamma.reshape(1, 1, D), beta.reshape(1, 1, D))
